# baseline (speedup 1.0000x reference)
import functools

import jax
import jax.numpy as jnp
from jax import lax
from jax.experimental import pallas as pl
from jax.experimental.pallas import tpu as pltpu
from jax.experimental.pallas import tpu_sc as plsc

_N = 10000
_E = 320000
_NODE_DIM = 128
_EDGE_DIM = 16
_HID = 64

_NC = 2
_NS = 16
_NW = _NC * _NS
_L = 16

_CHUNK = 128
_GR = _CHUNK // 2
_NCHUNKS = _E // _CHUNK
_BASE_CH = _NCHUNKS // _NW
_EXTRA = _NCHUNKS - _BASE_CH * _NW

_EB = 3200
_EBE = 2 * _EB

_ROWS_PER_TILE = _N // _NS
_ZROWS = 125


def _chunk_edge_bases(c):
    r0 = c * _GR
    b = r0 // _EB
    jj = r0 - b * _EB
    left = b * _EBE + jj
    right = left + _EB
    return left, right


def _mesh():
    return plsc.VectorSubcoreMesh(core_axis_name="c", subcore_axis_name="s")


def _prep_body(nodes_ref, wemb_ref, bemb_ref, wa_ref, wb_ref,
               h_ref, a_ref, b_ref):
    h = jnp.dot(nodes_ref[...], wemb_ref[...],
                preferred_element_type=jnp.float32) + bemb_ref[...]
    h_ref[...] = h
    a_ref[...] = jnp.dot(h, wa_ref[...],
                         preferred_element_type=jnp.float32).astype(jnp.bfloat16)
    b_ref[...] = jnp.dot(h, wb_ref[...],
                         preferred_element_type=jnp.float32).astype(jnp.bfloat16)


def _prep(nodes, wemb, bemb, wa, wb):
    bn = 1000
    grid = _N // bn
    return pl.pallas_call(
        _prep_body,
        grid=(grid,),
        in_specs=[
            pl.BlockSpec((bn, _NODE_DIM), lambda i: (i, 0)),
            pl.BlockSpec((_NODE_DIM, _HID), lambda i: (0, 0)),
            pl.BlockSpec((1, _HID), lambda i: (0, 0)),
            pl.BlockSpec((_HID, _HID), lambda i: (0, 0)),
            pl.BlockSpec((_HID, _HID), lambda i: (0, 0)),
        ],
        out_specs=[
            pl.BlockSpec((bn, _HID), lambda i: (i, 0)),
            pl.BlockSpec((bn, _HID), lambda i: (i, 0)),
            pl.BlockSpec((bn, _HID), lambda i: (i, 0)),
        ],
        out_shape=[jax.ShapeDtypeStruct((_N, _HID), jnp.float32),
                   jax.ShapeDtypeStruct((_N, _HID), jnp.bfloat16),
                   jax.ShapeDtypeStruct((_N, _HID), jnp.bfloat16)],
    )(nodes, wemb, bemb, wa, wb)


def _sc_gather_add(a_tab, b_tab, ei):
    nbuf = 3
    nt = _BASE_CH

    @functools.partial(
        pl.kernel,
        out_type=jax.ShapeDtypeStruct((_E // 2, 2 * _HID), jnp.bfloat16),
        mesh=_mesh(),
        scratch_types=(
            [pltpu.VMEM((_CHUNK,), jnp.int32)] * nbuf
            + [pltpu.VMEM((_CHUNK,), jnp.int32)] * nbuf
            + [pltpu.VMEM((_CHUNK, _HID), jnp.bfloat16)] * nbuf
            + [pltpu.VMEM((_CHUNK, _HID), jnp.bfloat16)] * nbuf
            + [pltpu.VMEM((_GR, 2 * _HID), jnp.bfloat16)] * nbuf
            + [pltpu.SemaphoreType.DMA] * (3 * nbuf)
        ),
        compiler_params=pltpu.CompilerParams(use_tc_tiling_on_sc=False),
    )
    def k(a_hbm, b_hbm, ei_hbm, g_hbm, *scr):
        sidx = scr[0:nbuf]
        eidx = scr[nbuf:2 * nbuf]
        ra = scr[2 * nbuf:3 * nbuf]
        rb = scr[3 * nbuf:4 * nbuf]
        go = scr[4 * nbuf:5 * nbuf]
        sem_i = scr[5 * nbuf:5 * nbuf + nbuf]
        sem_g = scr[5 * nbuf + nbuf:5 * nbuf + 2 * nbuf]
        sem_w = scr[5 * nbuf + 2 * nbuf:5 * nbuf + 3 * nbuf]
        wid = lax.axis_index("s") * _NC + lax.axis_index("c")

        def idx_copies(t, q):
            c = wid + t * _NW
            left, right = _chunk_edge_bases(c)
            return [
                pltpu.make_async_copy(ei_hbm.at[0, pl.ds(left, _GR)],
                                      sidx[q].at[pl.ds(0, _GR)], sem_i[q]),
                pltpu.make_async_copy(ei_hbm.at[0, pl.ds(right, _GR)],
                                      sidx[q].at[pl.ds(_GR, _GR)], sem_i[q]),
                pltpu.make_async_copy(ei_hbm.at[1, pl.ds(left, _GR)],
                                      eidx[q].at[pl.ds(0, _GR)], sem_i[q]),
                pltpu.make_async_copy(ei_hbm.at[1, pl.ds(right, _GR)],
                                      eidx[q].at[pl.ds(_GR, _GR)], sem_i[q]),
            ]

        def gath_copies(q):
            return [
                pltpu.make_async_copy(a_hbm.at[sidx[q]], ra[q], sem_g[q]),
                pltpu.make_async_copy(b_hbm.at[eidx[q]], rb[q], sem_g[q]),
            ]

        def wb_copy(t, q):
            c = wid + t * _NW
            return pltpu.make_async_copy(
                go[q], g_hbm.at[pl.ds(c * _GR, _GR), :], sem_w[q])

        def compute(q):
            lb = 2 * _L

            def row_body(i, c2):
                for half in range(2):
                    for j in range(_HID // lb):
                        src = pl.ds(j * lb, lb)
                        dst = pl.ds(half * _HID + j * lb, lb)
                        go[q][i, dst] = (ra[q][half * _GR + i, src]
                                        + rb[q][half * _GR + i, src])
                return c2
            lax.fori_loop(0, _GR, row_body, 0, unroll=4)

        for d in idx_copies(0, 0):
            d.start()
        for d in idx_copies(1, 1):
            d.start()
        for d in idx_copies(0, 0):
            d.wait()
        for d in gath_copies(0):
            d.start()

        def triple(p, carry):
            for u in range(nbuf):
                t = nbuf * p + u
                q = u
                q1 = (u + 1) % nbuf
                q2 = (u + 2) % nbuf

                def issue_next_gather():
                    for d in idx_copies(t + 1, q1):
                        d.wait()
                    for d in gath_copies(q1):
                        d.start()

                if u == nbuf - 1:
                    pl.when(p < (nt // nbuf) - 1)(issue_next_gather)
                else:
                    issue_next_gather()

                def issue_next_idx():
                    for d in idx_copies(t + 2, q2):
                        d.start()

                if u == 0:
                    issue_next_idx()
                else:
                    pl.when(p < (nt // nbuf) - 1)(issue_next_idx)

                for d in gath_copies(q):
                    d.wait()
                pl.when(p > 0)(lambda: wb_copy(t - nbuf, q).wait())
                compute(q)
                wb_copy(t, q).start()
            return carry

        lax.fori_loop(0, nt // nbuf, triple, 0)
        for u in range(nbuf):
            wb_copy(nt - nbuf + u, u).wait()

        @pl.when(wid < _EXTRA)
        def _():
            t = nt
            for d in idx_copies(t, 0):
                d.start()
            for d in idx_copies(t, 0):
                d.wait()
            for d in gath_copies(0):
                d.start()
            for d in gath_copies(0):
                d.wait()
            compute(0)
            wb_copy(t, 0).start()
            wb_copy(t, 0).wait()

    return k(a_tab, b_tab, ei)


def _edge_body(g_ref, eft_lo_ref, eft_hi_ref, wc_lo_ref, wc_hi_ref,
               be1_ref, we2_ref, be2_ref, m_ref):
    dn = (((0,), (0,)), ((), ()))
    c_lo = lax.dot_general(eft_lo_ref[...], wc_lo_ref[...], dn,
                           preferred_element_type=jnp.float32)
    c_hi = lax.dot_general(eft_hi_ref[...], wc_hi_ref[...], dn,
                           preferred_element_type=jnp.float32)
    u = g_ref[...].astype(jnp.float32) + c_lo + c_hi + be1_ref[...]
    u = u * jax.nn.sigmoid(u)
    v = jnp.dot(u, we2_ref[...], preferred_element_type=jnp.float32) + be2_ref[...]
    m_ref[...] = v * jax.nn.sigmoid(v)


def _edge_mlp(g2, eft, wc_lo, wc_hi, be1_2, we2_2, be2_2):
    e2 = _E // 2
    grid = e2 // _EB
    return pl.pallas_call(
        _edge_body,
        grid=(grid,),
        in_specs=[
            pl.BlockSpec((_EB, 2 * _HID), lambda i: (i, 0)),
            pl.BlockSpec((_EDGE_DIM, _EB), lambda i: (0, 2 * i)),
            pl.BlockSpec((_EDGE_DIM, _EB), lambda i: (0, 2 * i + 1)),
            pl.BlockSpec((_EDGE_DIM, 2 * _HID), lambda i: (0, 0)),
            pl.BlockSpec((_EDGE_DIM, 2 * _HID), lambda i: (0, 0)),
            pl.BlockSpec((1, 2 * _HID), lambda i: (0, 0)),
            pl.BlockSpec((2 * _HID, 2 * _HID), lambda i: (0, 0)),
            pl.BlockSpec((1, 2 * _HID), lambda i: (0, 0)),
        ],
        out_specs=pl.BlockSpec((_EB, 2 * _HID), lambda i: (i, 0)),
        out_shape=jax.ShapeDtypeStruct((e2, 2 * _HID), jnp.float32),
    )(g2, eft, eft, wc_lo, wc_hi, be1_2, we2_2, be2_2)


def _sc_scatter_add(m2, ei):
    nbuf = 3
    nt = _BASE_CH

    @functools.partial(
        pl.kernel,
        out_type=jax.ShapeDtypeStruct((_NC, _NS, _ROWS_PER_TILE, _HID),
                                      jnp.float32),
        mesh=_mesh(),
        scratch_types=(
            [pltpu.VMEM((_CHUNK,), jnp.int32)] * nbuf
            + [pltpu.VMEM((_GR, 2 * _HID), jnp.float32)] * nbuf
            + [pltpu.VMEM((_CHUNK, _HID), jnp.float32)] * nbuf
            + [pltpu.VMEM((_ZROWS, _HID), jnp.float32)]
            + [pltpu.VMEM_SHARED((_N, _HID), jnp.float32)]
            + [pltpu.SemaphoreType.DMA] * (2 * nbuf)
        ),
        compiler_params=pltpu.CompilerParams(use_tc_tiling_on_sc=False),
    )
    def k(m_hbm, ei_hbm, agg_hbm, *scr):
        idx = scr[0:nbuf]
        rows2 = scr[nbuf:2 * nbuf]
        rows = scr[2 * nbuf:3 * nbuf]
        zbuf = scr[3 * nbuf]
        shared = scr[3 * nbuf + 1]
        sem_l = scr[3 * nbuf + 2:3 * nbuf + 2 + nbuf]
        sem_s = scr[3 * nbuf + 2 + nbuf:3 * nbuf + 2 + 2 * nbuf]
        cid = lax.axis_index("c")
        sid = lax.axis_index("s")
        wid = sid * _NC + cid

        def load_copies(t, q):
            c = wid + t * _NW
            left, right = _chunk_edge_bases(c)
            return [
                pltpu.make_async_copy(ei_hbm.at[0, pl.ds(left, _GR)],
                                      idx[q].at[pl.ds(0, _GR)], sem_l[q]),
                pltpu.make_async_copy(ei_hbm.at[0, pl.ds(right, _GR)],
                                      idx[q].at[pl.ds(_GR, _GR)], sem_l[q]),
                pltpu.make_async_copy(m_hbm.at[pl.ds(c * _GR, _GR), :],
                                      rows2[q], sem_l[q]),
            ]

        def repack(q):
            def body(i, c2):
                for half in range(2):
                    for j in range(_HID // _L):
                        src = pl.ds(half * _HID + j * _L, _L)
                        dst = pl.ds(j * _L, _L)
                        rows[q][half * _GR + i, dst] = rows2[q][i, src]
                return c2
            lax.fori_loop(0, _GR, body, 0, unroll=4)

        def scat_wait(q):
            pltpu.make_async_copy(rows[q], shared.at[idx[q]], sem_s[q]).wait()

        def zb(i, c2):
            for j in range(_HID // _L):
                zbuf[i, pl.ds(j * _L, _L)] = jnp.zeros((_L,), jnp.float32)
            return c2

        lax.fori_loop(0, _ZROWS, zb, 0, unroll=4)
        for kk in range(_ROWS_PER_TILE // _ZROWS):
            pltpu.sync_copy(
                zbuf, shared.at[pl.ds(sid * _ROWS_PER_TILE + kk * _ZROWS,
                                      _ZROWS), :])
        plsc.subcore_barrier()

        for d in load_copies(0, 0):
            d.start()

        def triple(p, carry):
            for u in range(nbuf):
                t = nbuf * p + u
                q = u
                q1 = (u + 1) % nbuf

                def slot_free_then_load():
                    pl.when(t >= 2)(lambda: scat_wait(q1))
                    for d in load_copies(t + 1, q1):
                        d.start()

                if u == nbuf - 1:
                    pl.when(p < (nt // nbuf) - 1)(slot_free_then_load)
                else:
                    slot_free_then_load()

                for d in load_copies(t, q):
                    d.wait()
                repack(q)
                pltpu.async_copy(rows[q], shared.at[idx[q]], sem_s[q],
                                 add=True)
            return carry

        lax.fori_loop(0, nt // nbuf, triple, 0)
        for u in range(nbuf):
            scat_wait((nt - nbuf + u) % nbuf)

        @pl.when(wid < _EXTRA)
        def _():
            t = nt
            for d in load_copies(t, 0):
                d.start()
            for d in load_copies(t, 0):
                d.wait()
            repack(0)
            pltpu.async_copy(rows[0], shared.at[idx[0]], sem_s[0], add=True)
            scat_wait(0)

        plsc.subcore_barrier()

        for kk in range(_ROWS_PER_TILE // _ZROWS):
            sl = pl.ds(sid * _ROWS_PER_TILE + kk * _ZROWS, _ZROWS)
            pltpu.sync_copy(shared.at[sl, :],
                            agg_hbm.at[cid, sid, pl.ds(kk * _ZROWS, _ZROWS), :])

    return k(m2, ei)


def _final_body(h_ref, a0_ref, a1_ref, wn1h_ref, wn1a_ref, bn1_ref,
                wn2_ref, bn2_ref, wo_ref, bo_ref, out_ref, acc_ref):
    i = pl.program_id(0)
    h = h_ref[...]
    agg = a0_ref[0] + a1_ref[0]
    t = (jnp.dot(h, wn1h_ref[...], preferred_element_type=jnp.float32)
         + jnp.dot(agg, wn1a_ref[...], preferred_element_type=jnp.float32)
         + bn1_ref[...])
    t = t * jax.nn.sigmoid(t)
    hn = h + jnp.dot(t, wn2_ref[...],
                     preferred_element_type=jnp.float32) + bn2_ref[...]
    s = jnp.sum(hn, axis=0, keepdims=True)

    @pl.when(i == 0)
    def _():
        acc_ref[...] = s

    @pl.when(i > 0)
    def _():
        acc_ref[...] = acc_ref[...] + s

    @pl.when(i == pl.num_programs(0) - 1)
    def _():
        pooled = acc_ref[...] * (1.0 / _N)
        out_ref[...] = jnp.dot(pooled, wo_ref[...],
                               preferred_element_type=jnp.float32) + bo_ref[...]


def _final(h, aggp, wn1h, wn1a, bn1, wn2, bn2, wo, bo):
    bn = 1000
    grid = _N // bn
    return pl.pallas_call(
        _final_body,
        grid=(grid,),
        in_specs=[
            pl.BlockSpec((bn, _HID), lambda i: (i, 0)),
            pl.BlockSpec((1, bn, _HID), lambda i: (0, i, 0)),
            pl.BlockSpec((1, bn, _HID), lambda i: (1, i, 0)),
            pl.BlockSpec((_HID, _HID), lambda i: (0, 0)),
            pl.BlockSpec((_HID, _HID), lambda i: (0, 0)),
            pl.BlockSpec((1, _HID), lambda i: (0, 0)),
            pl.BlockSpec((_HID, _HID), lambda i: (0, 0)),
            pl.BlockSpec((1, _HID), lambda i: (0, 0)),
            pl.BlockSpec((_HID, 1), lambda i: (0, 0)),
            pl.BlockSpec((1, 1), lambda i: (0, 0)),
        ],
        out_specs=pl.BlockSpec((1, 1), lambda i: (0, 0)),
        out_shape=jax.ShapeDtypeStruct((1, 1), jnp.float32),
        scratch_shapes=[pltpu.VMEM((1, _HID), jnp.float32)],
    )(h, aggp, aggp, wn1h, wn1a, bn1, wn2, bn2, wo, bo)


def _blockdiag2(w):
    z = jnp.zeros_like(w)
    return jnp.concatenate(
        [jnp.concatenate([w, z], axis=1), jnp.concatenate([z, w], axis=1)],
        axis=0)


def kernel(nodes, edge_indices, edge_features, W_emb, b_emb, We1, be1,
           We2, be2, Wn1, bn1, Wn2, bn2, Wo, bo):
    ei = edge_indices.astype(jnp.int32)
    wa = We1[:_HID]
    wb = We1[_HID:2 * _HID]
    wc = We1[2 * _HID:]
    we2_2 = _blockdiag2(We2)
    zc = jnp.zeros_like(wc)
    wc_lo = jnp.concatenate([wc, zc], axis=1)
    wc_hi = jnp.concatenate([zc, wc], axis=1)
    be1_2 = jnp.tile(be1, 2).reshape(1, 2 * _HID)
    be2_2 = jnp.tile(be2, 2).reshape(1, 2 * _HID)
    h, a_tab, b_tab = _prep(nodes, W_emb, b_emb.reshape(1, _HID), wa, wb)
    g2 = _sc_gather_add(a_tab, b_tab, ei)
    m2 = _edge_mlp(g2, edge_features.T, wc_lo, wc_hi, be1_2, we2_2, be2_2)
    aggp = _sc_scatter_add(m2, ei)
    out = _final(h, aggp.reshape(_NC, _N, _HID),
                 Wn1[:_HID], Wn1[_HID:], bn1.reshape(1, _HID),
                 Wn2, bn2.reshape(1, _HID), Wo, bo.reshape(1, 1))
    return out.reshape((1,))


# speedup vs baseline: 1.3109x; 1.3109x over previous
import functools

import numpy as np

import jax
import jax.numpy as jnp
from jax import lax
from jax.experimental import pallas as pl
from jax.experimental.pallas import tpu as pltpu
from jax.experimental.pallas import tpu_sc as plsc

_N = 10000
_E = 320000
_NODE_DIM = 128
_EDGE_DIM = 16
_HID = 64

_NC = 2
_NS = 16
_NW = _NC * _NS
_L = 16

_CHUNK = 128
_GR = _CHUNK // 2
_NCHUNKS = _E // _CHUNK
_BASE_CH = _NCHUNKS // _NW
_EXTRA = _NCHUNKS - _BASE_CH * _NW

_EB = 3200
_EBE = 2 * _EB

_ROWS_PER_TILE = _N // _NS
_ZROWS = 125

_PERM64 = np.array([32 * k + off + j
                    for k in range(2) for j in range(16)
                    for off in (0, 16)])


def _chunk_edge_bases(c):
    r0 = c * _GR
    b = r0 // _EB
    jj = r0 - b * _EB
    left = b * _EBE + jj
    right = left + _EB
    return left, right


def _mesh():
    return plsc.VectorSubcoreMesh(core_axis_name="c", subcore_axis_name="s")


def _prep_body(nodes_ref, wemb_ref, bemb_ref, wa_ref, wb_ref,
               h_ref, a_ref, b_ref):
    h = jnp.dot(nodes_ref[...], wemb_ref[...],
                preferred_element_type=jnp.float32) + bemb_ref[...]
    h_ref[...] = h
    a_ref[...] = jnp.dot(h, wa_ref[...],
                         preferred_element_type=jnp.float32).astype(jnp.bfloat16)
    b_ref[...] = jnp.dot(h, wb_ref[...],
                         preferred_element_type=jnp.float32).astype(jnp.bfloat16)


def _prep(nodes, wemb, bemb, wa, wb):
    bn = 1000
    grid = _N // bn
    return pl.pallas_call(
        _prep_body,
        grid=(grid,),
        in_specs=[
            pl.BlockSpec((bn, _NODE_DIM), lambda i: (i, 0)),
            pl.BlockSpec((_NODE_DIM, _HID), lambda i: (0, 0)),
            pl.BlockSpec((1, _HID), lambda i: (0, 0)),
            pl.BlockSpec((_HID, _HID), lambda i: (0, 0)),
            pl.BlockSpec((_HID, _HID), lambda i: (0, 0)),
        ],
        out_specs=[
            pl.BlockSpec((bn, _HID), lambda i: (i, 0)),
            pl.BlockSpec((bn, _HID), lambda i: (i, 0)),
            pl.BlockSpec((bn, _HID), lambda i: (i, 0)),
        ],
        out_shape=[jax.ShapeDtypeStruct((_N, _HID), jnp.float32),
                   jax.ShapeDtypeStruct((_N, _HID), jnp.bfloat16),
                   jax.ShapeDtypeStruct((_N, _HID), jnp.bfloat16)],
    )(nodes, wemb, bemb, wa, wb)


def _sc_gather_add(a_tab, b_tab, ei):
    nbuf = 3
    nt = _BASE_CH

    @functools.partial(
        pl.kernel,
        out_type=jax.ShapeDtypeStruct((_E // 2, 2 * _HID), jnp.float32),
        mesh=_mesh(),
        scratch_types=(
            [pltpu.VMEM((_CHUNK,), jnp.int32)] * nbuf
            + [pltpu.VMEM((_CHUNK,), jnp.int32)] * nbuf
            + [pltpu.VMEM((_CHUNK, _HID), jnp.bfloat16)] * nbuf
            + [pltpu.VMEM((_CHUNK, _HID), jnp.bfloat16)] * nbuf
            + [pltpu.VMEM((_GR, 2 * _HID), jnp.float32)] * nbuf
            + [pltpu.SemaphoreType.DMA] * (3 * nbuf)
        ),
        compiler_params=pltpu.CompilerParams(use_tc_tiling_on_sc=False,
                                             needs_layout_passes=False),
    )
    def k(a_hbm, b_hbm, ei_hbm, g_hbm, *scr):
        sidx = scr[0:nbuf]
        eidx = scr[nbuf:2 * nbuf]
        ra = scr[2 * nbuf:3 * nbuf]
        rb = scr[3 * nbuf:4 * nbuf]
        go = scr[4 * nbuf:5 * nbuf]
        sem_i = scr[5 * nbuf:5 * nbuf + nbuf]
        sem_g = scr[5 * nbuf + nbuf:5 * nbuf + 2 * nbuf]
        sem_w = scr[5 * nbuf + 2 * nbuf:5 * nbuf + 3 * nbuf]
        wid = lax.axis_index("s") * _NC + lax.axis_index("c")

        def idx_copies(t, q):
            c = wid + t * _NW
            left, right = _chunk_edge_bases(c)
            return [
                pltpu.make_async_copy(ei_hbm.at[0, pl.ds(left, _GR)],
                                      sidx[q].at[pl.ds(0, _GR)], sem_i[q]),
                pltpu.make_async_copy(ei_hbm.at[0, pl.ds(right, _GR)],
                                      sidx[q].at[pl.ds(_GR, _GR)], sem_i[q]),
                pltpu.make_async_copy(ei_hbm.at[1, pl.ds(left, _GR)],
                                      eidx[q].at[pl.ds(0, _GR)], sem_i[q]),
                pltpu.make_async_copy(ei_hbm.at[1, pl.ds(right, _GR)],
                                      eidx[q].at[pl.ds(_GR, _GR)], sem_i[q]),
            ]

        def gath_copies(q):
            return [
                pltpu.make_async_copy(a_hbm.at[sidx[q]], ra[q], sem_g[q]),
                pltpu.make_async_copy(b_hbm.at[eidx[q]], rb[q], sem_g[q]),
            ]

        def wb_copy(t, q):
            c = wid + t * _NW
            return pltpu.make_async_copy(
                go[q], g_hbm.at[pl.ds(c * _GR, _GR), :], sem_w[q])

        def compute(q):
            lb = 2 * _L
            himask = jnp.int32(-65536)

            def row_body(i, c2):
                for half in range(2):
                    r = half * _GR + i
                    for k in range(_HID // lb):
                        src = pl.ds(k * lb, lb)
                        xa = plsc.bitcast(ra[q][r, src], jnp.int32)
                        xb = plsc.bitcast(rb[q][r, src], jnp.int32)
                        lo = (plsc.bitcast(xa << 16, jnp.float32)
                              + plsc.bitcast(xb << 16, jnp.float32))
                        hi = (plsc.bitcast(xa & himask, jnp.float32)
                              + plsc.bitcast(xb & himask, jnp.float32))
                        base = half * _HID + k * lb
                        go[q][i, pl.ds(base, _L)] = lo
                        go[q][i, pl.ds(base + _L, _L)] = hi
                return c2
            lax.fori_loop(0, _GR, row_body, 0, unroll=4)

        for d in idx_copies(0, 0):
            d.start()
        for d in idx_copies(1, 1):
            d.start()
        for d in idx_copies(0, 0):
            d.wait()
        for d in gath_copies(0):
            d.start()

        def triple(p, carry):
            for u in range(nbuf):
                t = nbuf * p + u
                q = u
                q1 = (u + 1) % nbuf
                q2 = (u + 2) % nbuf

                def issue_next_gather():
                    for d in idx_copies(t + 1, q1):
                        d.wait()
                    for d in gath_copies(q1):
                        d.start()

                if u == nbuf - 1:
                    pl.when(p < (nt // nbuf) - 1)(issue_next_gather)
                else:
                    issue_next_gather()

                def issue_next_idx():
                    for d in idx_copies(t + 2, q2):
                        d.start()

                if u == 0:
                    issue_next_idx()
                else:
                    pl.when(p < (nt // nbuf) - 1)(issue_next_idx)

                for d in gath_copies(q):
                    d.wait()
                pl.when(p > 0)(lambda: wb_copy(t - nbuf, q).wait())
                compute(q)
                wb_copy(t, q).start()
            return carry

        lax.fori_loop(0, nt // nbuf, triple, 0)
        for u in range(nbuf):
            wb_copy(nt - nbuf + u, u).wait()

        @pl.when(wid < _EXTRA)
        def _():
            t = nt
            for d in idx_copies(t, 0):
                d.start()
            for d in idx_copies(t, 0):
                d.wait()
            for d in gath_copies(0):
                d.start()
            for d in gath_copies(0):
                d.wait()
            compute(0)
            wb_copy(t, 0).start()
            wb_copy(t, 0).wait()

    return k(a_tab, b_tab, ei)


def _edge_body(g_ref, eft_lo_ref, eft_hi_ref, wc_lo_ref, wc_hi_ref,
               be1_ref, we2_ref, be2_ref, m_ref):
    dn = (((0,), (0,)), ((), ()))
    c_lo = lax.dot_general(eft_lo_ref[...], wc_lo_ref[...], dn,
                           preferred_element_type=jnp.float32)
    c_hi = lax.dot_general(eft_hi_ref[...], wc_hi_ref[...], dn,
                           preferred_element_type=jnp.float32)
    u = g_ref[...] + c_lo + c_hi + be1_ref[...]
    u = u * jax.nn.sigmoid(u)
    v = jnp.dot(u, we2_ref[...], preferred_element_type=jnp.float32) + be2_ref[...]
    m_ref[...] = v * jax.nn.sigmoid(v)


def _edge_mlp(g2, eft, wc_lo, wc_hi, be1_2, we2_2, be2_2):
    e2 = _E // 2
    grid = e2 // _EB
    return pl.pallas_call(
        _edge_body,
        grid=(grid,),
        in_specs=[
            pl.BlockSpec((_EB, 2 * _HID), lambda i: (i, 0)),
            pl.BlockSpec((_EDGE_DIM, _EB), lambda i: (0, 2 * i)),
            pl.BlockSpec((_EDGE_DIM, _EB), lambda i: (0, 2 * i + 1)),
            pl.BlockSpec((_EDGE_DIM, 2 * _HID), lambda i: (0, 0)),
            pl.BlockSpec((_EDGE_DIM, 2 * _HID), lambda i: (0, 0)),
            pl.BlockSpec((1, 2 * _HID), lambda i: (0, 0)),
            pl.BlockSpec((2 * _HID, 2 * _HID), lambda i: (0, 0)),
            pl.BlockSpec((1, 2 * _HID), lambda i: (0, 0)),
        ],
        out_specs=pl.BlockSpec((_EB, 2 * _HID), lambda i: (i, 0)),
        out_shape=jax.ShapeDtypeStruct((e2, 2 * _HID), jnp.float32),
    )(g2, eft, eft, wc_lo, wc_hi, be1_2, we2_2, be2_2)


def _sc_scatter_add(m2, ei):
    nbuf = 3
    nt = _BASE_CH

    @functools.partial(
        pl.kernel,
        out_type=jax.ShapeDtypeStruct((_NC, _NS, _ROWS_PER_TILE, _HID),
                                      jnp.float32),
        mesh=_mesh(),
        scratch_types=(
            [pltpu.VMEM((_CHUNK,), jnp.int32)] * nbuf
            + [pltpu.VMEM((_GR, 2 * _HID), jnp.float32)] * nbuf
            + [pltpu.VMEM((_CHUNK, _HID), jnp.float32)] * nbuf
            + [pltpu.VMEM((_ZROWS, _HID), jnp.float32)]
            + [pltpu.VMEM_SHARED((_N, _HID), jnp.float32)]
            + [pltpu.SemaphoreType.DMA] * (2 * nbuf)
        ),
        compiler_params=pltpu.CompilerParams(use_tc_tiling_on_sc=False),
    )
    def k(m_hbm, ei_hbm, agg_hbm, *scr):
        idx = scr[0:nbuf]
        rows2 = scr[nbuf:2 * nbuf]
        rows = scr[2 * nbuf:3 * nbuf]
        zbuf = scr[3 * nbuf]
        shared = scr[3 * nbuf + 1]
        sem_l = scr[3 * nbuf + 2:3 * nbuf + 2 + nbuf]
        sem_s = scr[3 * nbuf + 2 + nbuf:3 * nbuf + 2 + 2 * nbuf]
        cid = lax.axis_index("c")
        sid = lax.axis_index("s")
        wid = sid * _NC + cid

        def load_copies(t, q):
            c = wid + t * _NW
            left, right = _chunk_edge_bases(c)
            return [
                pltpu.make_async_copy(ei_hbm.at[0, pl.ds(left, _GR)],
                                      idx[q].at[pl.ds(0, _GR)], sem_l[q]),
                pltpu.make_async_copy(ei_hbm.at[0, pl.ds(right, _GR)],
                                      idx[q].at[pl.ds(_GR, _GR)], sem_l[q]),
                pltpu.make_async_copy(m_hbm.at[pl.ds(c * _GR, _GR), :],
                                      rows2[q], sem_l[q]),
            ]

        def repack(q):
            def body(i, c2):
                for half in range(2):
                    for j in range(_HID // _L):
                        src = pl.ds(half * _HID + j * _L, _L)
                        dst = pl.ds(j * _L, _L)
                        rows[q][half * _GR + i, dst] = rows2[q][i, src]
                return c2
            lax.fori_loop(0, _GR, body, 0, unroll=4)

        def scat_wait(q):
            pltpu.make_async_copy(rows[q], shared.at[idx[q]], sem_s[q]).wait()

        def zb(i, c2):
            for j in range(_HID // _L):
                zbuf[i, pl.ds(j * _L, _L)] = jnp.zeros((_L,), jnp.float32)
            return c2

        lax.fori_loop(0, _ZROWS, zb, 0, unroll=4)
        for kk in range(_ROWS_PER_TILE // _ZROWS):
            pltpu.sync_copy(
                zbuf, shared.at[pl.ds(sid * _ROWS_PER_TILE + kk * _ZROWS,
                                      _ZROWS), :])
        plsc.subcore_barrier()

        for d in load_copies(0, 0):
            d.start()

        def triple(p, carry):
            for u in range(nbuf):
                t = nbuf * p + u
                q = u
                q1 = (u + 1) % nbuf

                def slot_free_then_load():
                    pl.when(t >= 2)(lambda: scat_wait(q1))
                    for d in load_copies(t + 1, q1):
                        d.start()

                if u == nbuf - 1:
                    pl.when(p < (nt // nbuf) - 1)(slot_free_then_load)
                else:
                    slot_free_then_load()

                for d in load_copies(t, q):
                    d.wait()
                repack(q)
                pltpu.async_copy(rows[q], shared.at[idx[q]], sem_s[q],
                                 add=True)
            return carry

        lax.fori_loop(0, nt // nbuf, triple, 0)
        for u in range(nbuf):
            scat_wait((nt - nbuf + u) % nbuf)

        @pl.when(wid < _EXTRA)
        def _():
            t = nt
            for d in load_copies(t, 0):
                d.start()
            for d in load_copies(t, 0):
                d.wait()
            repack(0)
            pltpu.async_copy(rows[0], shared.at[idx[0]], sem_s[0], add=True)
            scat_wait(0)

        plsc.subcore_barrier()

        for kk in range(_ROWS_PER_TILE // _ZROWS):
            sl = pl.ds(sid * _ROWS_PER_TILE + kk * _ZROWS, _ZROWS)
            pltpu.sync_copy(shared.at[sl, :],
                            agg_hbm.at[cid, sid, pl.ds(kk * _ZROWS, _ZROWS), :])

    return k(m2, ei)


def _final_body(h_ref, a0_ref, a1_ref, wn1h_ref, wn1a_ref, bn1_ref,
                wn2_ref, bn2_ref, wo_ref, bo_ref, out_ref, acc_ref):
    i = pl.program_id(0)
    h = h_ref[...]
    agg = a0_ref[0] + a1_ref[0]
    t = (jnp.dot(h, wn1h_ref[...], preferred_element_type=jnp.float32)
         + jnp.dot(agg, wn1a_ref[...], preferred_element_type=jnp.float32)
         + bn1_ref[...])
    t = t * jax.nn.sigmoid(t)
    hn = h + jnp.dot(t, wn2_ref[...],
                     preferred_element_type=jnp.float32) + bn2_ref[...]
    s = jnp.sum(hn, axis=0, keepdims=True)

    @pl.when(i == 0)
    def _():
        acc_ref[...] = s

    @pl.when(i > 0)
    def _():
        acc_ref[...] = acc_ref[...] + s

    @pl.when(i == pl.num_programs(0) - 1)
    def _():
        pooled = acc_ref[...] * (1.0 / _N)
        out_ref[...] = jnp.dot(pooled, wo_ref[...],
                               preferred_element_type=jnp.float32) + bo_ref[...]


def _final(h, aggp, wn1h, wn1a, bn1, wn2, bn2, wo, bo):
    bn = 1000
    grid = _N // bn
    return pl.pallas_call(
        _final_body,
        grid=(grid,),
        in_specs=[
            pl.BlockSpec((bn, _HID), lambda i: (i, 0)),
            pl.BlockSpec((1, bn, _HID), lambda i: (0, i, 0)),
            pl.BlockSpec((1, bn, _HID), lambda i: (1, i, 0)),
            pl.BlockSpec((_HID, _HID), lambda i: (0, 0)),
            pl.BlockSpec((_HID, _HID), lambda i: (0, 0)),
            pl.BlockSpec((1, _HID), lambda i: (0, 0)),
            pl.BlockSpec((_HID, _HID), lambda i: (0, 0)),
            pl.BlockSpec((1, _HID), lambda i: (0, 0)),
            pl.BlockSpec((_HID, 1), lambda i: (0, 0)),
            pl.BlockSpec((1, 1), lambda i: (0, 0)),
        ],
        out_specs=pl.BlockSpec((1, 1), lambda i: (0, 0)),
        out_shape=jax.ShapeDtypeStruct((1, 1), jnp.float32),
        scratch_shapes=[pltpu.VMEM((1, _HID), jnp.float32)],
    )(h, aggp, aggp, wn1h, wn1a, bn1, wn2, bn2, wo, bo)


def _blockdiag2(w):
    z = jnp.zeros_like(w)
    return jnp.concatenate(
        [jnp.concatenate([w, z], axis=1), jnp.concatenate([z, w], axis=1)],
        axis=0)


def kernel(nodes, edge_indices, edge_features, W_emb, b_emb, We1, be1,
           We2, be2, Wn1, bn1, Wn2, bn2, Wo, bo):
    ei = edge_indices.astype(jnp.int32)
    wa = We1[:_HID][:, _PERM64]
    wb = We1[_HID:2 * _HID][:, _PERM64]
    wc = We1[2 * _HID:]
    we2_2 = _blockdiag2(We2)
    zc = jnp.zeros_like(wc)
    wc_lo = jnp.concatenate([wc, zc], axis=1)
    wc_hi = jnp.concatenate([zc, wc], axis=1)
    be1_2 = jnp.tile(be1, 2).reshape(1, 2 * _HID)
    be2_2 = jnp.tile(be2, 2).reshape(1, 2 * _HID)
    h, a_tab, b_tab = _prep(nodes, W_emb, b_emb.reshape(1, _HID), wa, wb)
    g2 = _sc_gather_add(a_tab, b_tab, ei)
    m2 = _edge_mlp(g2, edge_features.T, wc_lo, wc_hi, be1_2, we2_2, be2_2)
    aggp = _sc_scatter_add(m2, ei)
    out = _final(h, aggp.reshape(_NC, _N, _HID),
                 Wn1[:_HID], Wn1[_HID:], bn1.reshape(1, _HID),
                 Wn2, bn2.reshape(1, _HID), Wo, bo.reshape(1, 1))
    return out.reshape((1,))


# speedup vs baseline: 1.4994x; 1.1438x over previous
import functools

import numpy as np

import jax
import jax.numpy as jnp
from jax import lax
from jax.experimental import pallas as pl
from jax.experimental.pallas import tpu as pltpu
from jax.experimental.pallas import tpu_sc as plsc

_N = 10000
_E = 320000
_NODE_DIM = 128
_EDGE_DIM = 16
_HID = 64

_NC = 2
_NS = 16
_NW = _NC * _NS
_L = 16

_CHUNK = 128
_GR = _CHUNK // 2
_NCHUNKS = _E // _CHUNK
_NSLAB = 2
_NCH_S = _NCHUNKS // _NSLAB
_BASE_CH = _NCH_S // _NW
_EXTRA = _NCH_S - _BASE_CH * _NW

_EB = 3200
_EBE = 2 * _EB

_ROWS_PER_TILE = _N // _NS
_ZROWS = 125

_PERM64 = np.array([32 * k + off + j
                    for k in range(2) for j in range(16)
                    for off in (0, 16)])


def _chunk_edge_bases(c):
    r0 = c * _GR
    b = r0 // _EB
    jj = r0 - b * _EB
    left = b * _EBE + jj
    right = left + _EB
    return left, right


def _mesh():
    return plsc.VectorSubcoreMesh(core_axis_name="c", subcore_axis_name="s")


def _prep_body(nodes_ref, wemb_ref, bemb_ref, wa_ref, wb_ref,
               h_ref, a_ref, b_ref):
    h = jnp.dot(nodes_ref[...], wemb_ref[...],
                preferred_element_type=jnp.float32) + bemb_ref[...]
    h_ref[...] = h
    a_ref[...] = jnp.dot(h, wa_ref[...],
                         preferred_element_type=jnp.float32).astype(jnp.bfloat16)
    b_ref[...] = jnp.dot(h, wb_ref[...],
                         preferred_element_type=jnp.float32).astype(jnp.bfloat16)


def _prep(nodes, wemb, bemb, wa, wb):
    bn = 1000
    grid = _N // bn
    return pl.pallas_call(
        _prep_body,
        grid=(grid,),
        in_specs=[
            pl.BlockSpec((bn, _NODE_DIM), lambda i: (i, 0)),
            pl.BlockSpec((_NODE_DIM, _HID), lambda i: (0, 0)),
            pl.BlockSpec((1, _HID), lambda i: (0, 0)),
            pl.BlockSpec((_HID, _HID), lambda i: (0, 0)),
            pl.BlockSpec((_HID, _HID), lambda i: (0, 0)),
        ],
        out_specs=[
            pl.BlockSpec((bn, _HID), lambda i: (i, 0)),
            pl.BlockSpec((bn, _HID), lambda i: (i, 0)),
            pl.BlockSpec((bn, _HID), lambda i: (i, 0)),
        ],
        out_shape=[jax.ShapeDtypeStruct((_N, _HID), jnp.float32),
                   jax.ShapeDtypeStruct((_N, _HID), jnp.bfloat16),
                   jax.ShapeDtypeStruct((_N, _HID), jnp.bfloat16)],
    )(nodes, wemb, bemb, wa, wb)


def _sc_gather_add(a_tab, b_tab, ei, slab):
    nbuf = 3
    nt = _BASE_CH
    c0 = slab * _NCH_S

    @functools.partial(
        pl.kernel,
        out_type=jax.ShapeDtypeStruct((_E // (2 * _NSLAB), 2 * _HID),
                                      jnp.float32),
        mesh=_mesh(),
        scratch_types=(
            [pltpu.VMEM((_CHUNK,), jnp.int32)] * nbuf
            + [pltpu.VMEM((_CHUNK,), jnp.int32)] * nbuf
            + [pltpu.VMEM((_CHUNK, _HID), jnp.bfloat16)] * nbuf
            + [pltpu.VMEM((_CHUNK, _HID), jnp.bfloat16)] * nbuf
            + [pltpu.VMEM((_GR, 2 * _HID), jnp.float32)] * nbuf
            + [pltpu.SemaphoreType.DMA] * (3 * nbuf)
        ),
        compiler_params=pltpu.CompilerParams(use_tc_tiling_on_sc=False,
                                             needs_layout_passes=False),
    )
    def k(a_hbm, b_hbm, ei_hbm, g_hbm, *scr):
        sidx = scr[0:nbuf]
        eidx = scr[nbuf:2 * nbuf]
        ra = scr[2 * nbuf:3 * nbuf]
        rb = scr[3 * nbuf:4 * nbuf]
        go = scr[4 * nbuf:5 * nbuf]
        sem_i = scr[5 * nbuf:5 * nbuf + nbuf]
        sem_g = scr[5 * nbuf + nbuf:5 * nbuf + 2 * nbuf]
        sem_w = scr[5 * nbuf + 2 * nbuf:5 * nbuf + 3 * nbuf]
        wid = lax.axis_index("s") * _NC + lax.axis_index("c")

        def idx_copies(t, q):
            c = c0 + wid + t * _NW
            left, right = _chunk_edge_bases(c)
            return [
                pltpu.make_async_copy(ei_hbm.at[0, pl.ds(left, _GR)],
                                      sidx[q].at[pl.ds(0, _GR)], sem_i[q]),
                pltpu.make_async_copy(ei_hbm.at[0, pl.ds(right, _GR)],
                                      sidx[q].at[pl.ds(_GR, _GR)], sem_i[q]),
                pltpu.make_async_copy(ei_hbm.at[1, pl.ds(left, _GR)],
                                      eidx[q].at[pl.ds(0, _GR)], sem_i[q]),
                pltpu.make_async_copy(ei_hbm.at[1, pl.ds(right, _GR)],
                                      eidx[q].at[pl.ds(_GR, _GR)], sem_i[q]),
            ]

        def gath_copies(q):
            return [
                pltpu.make_async_copy(a_hbm.at[sidx[q]], ra[q], sem_g[q]),
                pltpu.make_async_copy(b_hbm.at[eidx[q]], rb[q], sem_g[q]),
            ]

        def wb_copy(t, q):
            cs = wid + t * _NW
            return pltpu.make_async_copy(
                go[q], g_hbm.at[pl.ds(cs * _GR, _GR), :], sem_w[q])

        def compute(q):
            lb = 2 * _L
            himask = jnp.int32(-65536)

            def row_body(i, c2):
                for half in range(2):
                    r = half * _GR + i
                    for k in range(_HID // lb):
                        src = pl.ds(k * lb, lb)
                        xa = plsc.bitcast(ra[q][r, src], jnp.int32)
                        xb = plsc.bitcast(rb[q][r, src], jnp.int32)
                        lo = (plsc.bitcast(xa << 16, jnp.float32)
                              + plsc.bitcast(xb << 16, jnp.float32))
                        hi = (plsc.bitcast(xa & himask, jnp.float32)
                              + plsc.bitcast(xb & himask, jnp.float32))
                        base = half * _HID + k * lb
                        go[q][i, pl.ds(base, _L)] = lo
                        go[q][i, pl.ds(base + _L, _L)] = hi
                return c2
            lax.fori_loop(0, _GR, row_body, 0, unroll=4)

        for d in idx_copies(0, 0):
            d.start()
        for d in idx_copies(1, 1):
            d.start()
        for d in idx_copies(0, 0):
            d.wait()
        for d in gath_copies(0):
            d.start()

        def triple(p, carry):
            for u in range(nbuf):
                t = nbuf * p + u
                q = u
                q1 = (u + 1) % nbuf
                q2 = (u + 2) % nbuf

                def issue_next_gather():
                    for d in idx_copies(t + 1, q1):
                        d.wait()
                    for d in gath_copies(q1):
                        d.start()

                if u == nbuf - 1:
                    pl.when(p < (nt // nbuf) - 1)(issue_next_gather)
                else:
                    issue_next_gather()

                def issue_next_idx():
                    for d in idx_copies(t + 2, q2):
                        d.start()

                if u == 0:
                    issue_next_idx()
                else:
                    pl.when(p < (nt // nbuf) - 1)(issue_next_idx)

                for d in gath_copies(q):
                    d.wait()
                pl.when(p > 0)(lambda: wb_copy(t - nbuf, q).wait())
                compute(q)
                wb_copy(t, q).start()
            return carry

        lax.fori_loop(0, nt // nbuf, triple, 0)
        for u in range(nbuf):
            wb_copy(nt - nbuf + u, u).wait()

        @pl.when(wid < _EXTRA)
        def _():
            t = nt
            for d in idx_copies(t, 0):
                d.start()
            for d in idx_copies(t, 0):
                d.wait()
            for d in gath_copies(0):
                d.start()
            for d in gath_copies(0):
                d.wait()
            compute(0)
            wb_copy(t, 0).start()
            wb_copy(t, 0).wait()

    return k(a_tab, b_tab, ei)


def _edge_body(g_ref, eft_lo_ref, eft_hi_ref, wc_lo_ref, wc_hi_ref,
               be1_ref, we2_ref, be2_ref, m_ref):
    dn = (((0,), (0,)), ((), ()))
    c_lo = lax.dot_general(eft_lo_ref[...], wc_lo_ref[...], dn,
                           preferred_element_type=jnp.float32)
    c_hi = lax.dot_general(eft_hi_ref[...], wc_hi_ref[...], dn,
                           preferred_element_type=jnp.float32)
    u = g_ref[...] + c_lo + c_hi + be1_ref[...]
    u = u * jax.nn.sigmoid(u)
    v = jnp.dot(u, we2_ref[...], preferred_element_type=jnp.float32) + be2_ref[...]
    m_ref[...] = v * jax.nn.sigmoid(v)


def _edge_mlp(g2, eft, wc_lo, wc_hi, be1_2, we2_2, be2_2, slab):
    e2 = _E // (2 * _NSLAB)
    grid = e2 // _EB
    boff = 2 * slab * grid
    return pl.pallas_call(
        _edge_body,
        grid=(grid,),
        in_specs=[
            pl.BlockSpec((_EB, 2 * _HID), lambda i: (i, 0)),
            pl.BlockSpec((_EDGE_DIM, _EB), lambda i: (0, boff + 2 * i)),
            pl.BlockSpec((_EDGE_DIM, _EB), lambda i: (0, boff + 2 * i + 1)),
            pl.BlockSpec((_EDGE_DIM, 2 * _HID), lambda i: (0, 0)),
            pl.BlockSpec((_EDGE_DIM, 2 * _HID), lambda i: (0, 0)),
            pl.BlockSpec((1, 2 * _HID), lambda i: (0, 0)),
            pl.BlockSpec((2 * _HID, 2 * _HID), lambda i: (0, 0)),
            pl.BlockSpec((1, 2 * _HID), lambda i: (0, 0)),
        ],
        out_specs=pl.BlockSpec((_EB, 2 * _HID), lambda i: (i, 0)),
        out_shape=jax.ShapeDtypeStruct((e2, 2 * _HID), jnp.float32),
        name=f"edge_mlp_s{slab}",
    )(g2, eft, eft, wc_lo, wc_hi, be1_2, we2_2, be2_2)


def _sc_scatter_add(m2, ei, slab):
    nbuf = 3
    nt = _BASE_CH
    c0 = slab * _NCH_S

    @functools.partial(
        pl.kernel,
        out_type=jax.ShapeDtypeStruct((_NC, _NS, _ROWS_PER_TILE, _HID),
                                      jnp.float32),
        mesh=_mesh(),
        scratch_types=(
            [pltpu.VMEM((_CHUNK,), jnp.int32)] * nbuf
            + [pltpu.VMEM((_GR, 2 * _HID), jnp.float32)] * nbuf
            + [pltpu.VMEM((_CHUNK, _HID), jnp.float32)] * nbuf
            + [pltpu.VMEM((_ZROWS, _HID), jnp.float32)]
            + [pltpu.VMEM_SHARED((_N, _HID), jnp.float32)]
            + [pltpu.SemaphoreType.DMA] * (2 * nbuf)
        ),
        compiler_params=pltpu.CompilerParams(use_tc_tiling_on_sc=False),
    )
    def k(m_hbm, ei_hbm, agg_hbm, *scr):
        idx = scr[0:nbuf]
        rows2 = scr[nbuf:2 * nbuf]
        rows = scr[2 * nbuf:3 * nbuf]
        zbuf = scr[3 * nbuf]
        shared = scr[3 * nbuf + 1]
        sem_l = scr[3 * nbuf + 2:3 * nbuf + 2 + nbuf]
        sem_s = scr[3 * nbuf + 2 + nbuf:3 * nbuf + 2 + 2 * nbuf]
        cid = lax.axis_index("c")
        sid = lax.axis_index("s")
        wid = sid * _NC + cid

        def load_copies(t, q):
            cs = wid + t * _NW
            left, right = _chunk_edge_bases(c0 + cs)
            return [
                pltpu.make_async_copy(ei_hbm.at[0, pl.ds(left, _GR)],
                                      idx[q].at[pl.ds(0, _GR)], sem_l[q]),
                pltpu.make_async_copy(ei_hbm.at[0, pl.ds(right, _GR)],
                                      idx[q].at[pl.ds(_GR, _GR)], sem_l[q]),
                pltpu.make_async_copy(m_hbm.at[pl.ds(cs * _GR, _GR), :],
                                      rows2[q], sem_l[q]),
            ]

        def repack(q):
            def body(i, c2):
                for half in range(2):
                    for j in range(_HID // _L):
                        src = pl.ds(half * _HID + j * _L, _L)
                        dst = pl.ds(j * _L, _L)
                        rows[q][half * _GR + i, dst] = rows2[q][i, src]
                return c2
            lax.fori_loop(0, _GR, body, 0, unroll=4)

        def scat_wait(q):
            pltpu.make_async_copy(rows[q], shared.at[idx[q]], sem_s[q]).wait()

        def zb(i, c2):
            for j in range(_HID // _L):
                zbuf[i, pl.ds(j * _L, _L)] = jnp.zeros((_L,), jnp.float32)
            return c2

        lax.fori_loop(0, _ZROWS, zb, 0, unroll=4)
        for kk in range(_ROWS_PER_TILE // _ZROWS):
            pltpu.sync_copy(
                zbuf, shared.at[pl.ds(sid * _ROWS_PER_TILE + kk * _ZROWS,
                                      _ZROWS), :])
        plsc.subcore_barrier()

        for d in load_copies(0, 0):
            d.start()

        def triple(p, carry):
            for u in range(nbuf):
                t = nbuf * p + u
                q = u
                q1 = (u + 1) % nbuf

                def slot_free_then_load():
                    pl.when(t >= 2)(lambda: scat_wait(q1))
                    for d in load_copies(t + 1, q1):
                        d.start()

                if u == nbuf - 1:
                    pl.when(p < (nt // nbuf) - 1)(slot_free_then_load)
                else:
                    slot_free_then_load()

                for d in load_copies(t, q):
                    d.wait()
                repack(q)
                pltpu.async_copy(rows[q], shared.at[idx[q]], sem_s[q],
                                 add=True)
            return carry

        lax.fori_loop(0, nt // nbuf, triple, 0)
        for u in range(nbuf):
            scat_wait((nt - nbuf + u) % nbuf)

        @pl.when(wid < _EXTRA)
        def _():
            t = nt
            for d in load_copies(t, 0):
                d.start()
            for d in load_copies(t, 0):
                d.wait()
            repack(0)
            pltpu.async_copy(rows[0], shared.at[idx[0]], sem_s[0], add=True)
            scat_wait(0)

        plsc.subcore_barrier()

        for kk in range(_ROWS_PER_TILE // _ZROWS):
            sl = pl.ds(sid * _ROWS_PER_TILE + kk * _ZROWS, _ZROWS)
            pltpu.sync_copy(shared.at[sl, :],
                            agg_hbm.at[cid, sid, pl.ds(kk * _ZROWS, _ZROWS), :])

    return k(m2, ei)


def _final_body(h_ref, a0_ref, a1_ref, a2_ref, a3_ref, wn1h_ref, wn1a_ref,
                bn1_ref, wn2_ref, bn2_ref, wo_ref, bo_ref, out_ref, acc_ref):
    i = pl.program_id(0)
    h = h_ref[...]
    agg = (a0_ref[0] + a1_ref[0]) + (a2_ref[0] + a3_ref[0])
    t = (jnp.dot(h, wn1h_ref[...], preferred_element_type=jnp.float32)
         + jnp.dot(agg, wn1a_ref[...], preferred_element_type=jnp.float32)
         + bn1_ref[...])
    t = t * jax.nn.sigmoid(t)
    hn = h + jnp.dot(t, wn2_ref[...],
                     preferred_element_type=jnp.float32) + bn2_ref[...]
    s = jnp.sum(hn, axis=0, keepdims=True)

    @pl.when(i == 0)
    def _():
        acc_ref[...] = s

    @pl.when(i > 0)
    def _():
        acc_ref[...] = acc_ref[...] + s

    @pl.when(i == pl.num_programs(0) - 1)
    def _():
        pooled = acc_ref[...] * (1.0 / _N)
        out_ref[...] = jnp.dot(pooled, wo_ref[...],
                               preferred_element_type=jnp.float32) + bo_ref[...]


def _final(h, aggp0, aggp1, wn1h, wn1a, bn1, wn2, bn2, wo, bo):
    bn = 1000
    grid = _N // bn
    return pl.pallas_call(
        _final_body,
        grid=(grid,),
        in_specs=[
            pl.BlockSpec((bn, _HID), lambda i: (i, 0)),
            pl.BlockSpec((1, bn, _HID), lambda i: (0, i, 0)),
            pl.BlockSpec((1, bn, _HID), lambda i: (1, i, 0)),
            pl.BlockSpec((1, bn, _HID), lambda i: (0, i, 0)),
            pl.BlockSpec((1, bn, _HID), lambda i: (1, i, 0)),
            pl.BlockSpec((_HID, _HID), lambda i: (0, 0)),
            pl.BlockSpec((_HID, _HID), lambda i: (0, 0)),
            pl.BlockSpec((1, _HID), lambda i: (0, 0)),
            pl.BlockSpec((_HID, _HID), lambda i: (0, 0)),
            pl.BlockSpec((1, _HID), lambda i: (0, 0)),
            pl.BlockSpec((_HID, 1), lambda i: (0, 0)),
            pl.BlockSpec((1, 1), lambda i: (0, 0)),
        ],
        out_specs=pl.BlockSpec((1, 1), lambda i: (0, 0)),
        out_shape=jax.ShapeDtypeStruct((1, 1), jnp.float32),
        scratch_shapes=[pltpu.VMEM((1, _HID), jnp.float32)],
    )(h, aggp0, aggp0, aggp1, aggp1, wn1h, wn1a, bn1, wn2, bn2, wo, bo)


def _blockdiag2(w):
    z = jnp.zeros_like(w)
    return jnp.concatenate(
        [jnp.concatenate([w, z], axis=1), jnp.concatenate([z, w], axis=1)],
        axis=0)


def kernel(nodes, edge_indices, edge_features, W_emb, b_emb, We1, be1,
           We2, be2, Wn1, bn1, Wn2, bn2, Wo, bo):
    ei = edge_indices.astype(jnp.int32)
    wa = We1[:_HID][:, _PERM64]
    wb = We1[_HID:2 * _HID][:, _PERM64]
    wc = We1[2 * _HID:]
    we2_2 = _blockdiag2(We2)
    zc = jnp.zeros_like(wc)
    wc_lo = jnp.concatenate([wc, zc], axis=1)
    wc_hi = jnp.concatenate([zc, wc], axis=1)
    be1_2 = jnp.tile(be1, 2).reshape(1, 2 * _HID)
    be2_2 = jnp.tile(be2, 2).reshape(1, 2 * _HID)
    h, a_tab, b_tab = _prep(nodes, W_emb, b_emb.reshape(1, _HID), wa, wb)
    eft = edge_features.T
    aggp = []
    for s in range(_NSLAB):
        g2 = _sc_gather_add(a_tab, b_tab, ei, s)
        m2 = _edge_mlp(g2, eft, wc_lo, wc_hi, be1_2, we2_2, be2_2, s)
        aggp.append(_sc_scatter_add(m2, ei, s).reshape(_NC, _N, _HID))
    out = _final(h, aggp[0], aggp[1],
                 Wn1[:_HID], Wn1[_HID:], bn1.reshape(1, _HID),
                 Wn2, bn2.reshape(1, _HID), Wo, bo.reshape(1, 1))
    return out.reshape((1,))
